# Initial kernel scaffold; baseline (speedup 1.0000x reference)
import jax
import jax.numpy as jnp
from jax.experimental import pallas as pl


def kernel(x, edge_index, edge_index_2_hop, edge_index_knn, g1_W, g1_as, g1_ad, g1_b, g2_W, g2_as, g2_ad, g2_b, g3_W, g3_as, g3_ad, g3_b, g4_W, g4_as, g4_ad, g4_b, g5_W, g5_as, g5_ad, g5_b, g6_W, g6_as, g6_ad, g6_b, l1_W, l1_b, l2_W, l2_b, l3_W, l3_b, agg_W):
    raise NotImplementedError("write your pallas kernel here")


# speedup vs baseline: 1.0279x; 1.0279x over previous
import jax
import jax.numpy as jnp
from jax.experimental import pallas as pl

N = 10000
HID = 128
C = 40
H = 8


def _gat(x, ei, W, a_s, a_d, b, f_out):
    n = x.shape[0]
    loop = jnp.arange(n, dtype=ei.dtype)
    src = jnp.concatenate([ei[0], loop])
    dst = jnp.concatenate([ei[1], loop])
    h = (x @ W).reshape(n, H, f_out)
    alpha_src = jnp.sum(h * a_s[None, :, :], axis=-1)
    alpha_dst = jnp.sum(h * a_d[None, :, :], axis=-1)
    e = jax.nn.leaky_relu(alpha_src[src] + alpha_dst[dst], 0.2)
    ex = jnp.exp(e)
    s = jax.ops.segment_sum(ex, dst, num_segments=n)
    alpha = ex / (s[dst] + 1e-16)
    out = jax.ops.segment_sum(h[src] * alpha[:, :, None], dst, num_segments=n)
    return out.mean(axis=1) + b


def _epilogue_body(h1_ref, h2_ref, h3_ref, l1W_ref, l1b_ref, l2W_ref, l2b_ref,
                   l3W_ref, l3b_ref, aggW_ref, out_ref):
    h1 = h1_ref[...]
    h2 = h2_ref[...]
    h3 = h3_ref[...]
    aggW = aggW_ref[...]
    a1 = jnp.tanh(h1 @ l1W_ref[...] + l1b_ref[...]) @ aggW
    a2 = jnp.tanh(h2 @ l2W_ref[...] + l2b_ref[...]) @ aggW
    a3 = jnp.tanh(h3 @ l3W_ref[...] + l3b_ref[...]) @ aggW
    m = jnp.maximum(jnp.maximum(a1, a2), a3)
    e1 = jnp.exp(a1 - m)
    e2 = jnp.exp(a2 - m)
    e3 = jnp.exp(a3 - m)
    tot = e1 + e2 + e3
    h = (e1 * h1 + e2 * h2 + e3 * h3) / tot
    hm = jnp.max(h, axis=1, keepdims=True)
    lse = jnp.log(jnp.sum(jnp.exp(h - hm), axis=1, keepdims=True)) + hm
    out_ref[...] = h - lse


def _epilogue(h1, h2, h3, l1_W, l1_b, l2_W, l2_b, l3_W, l3_b, agg_W):
    n = h1.shape[0]
    blk = 1000
    grid = (n // blk,)
    row_spec = pl.BlockSpec((blk, C), lambda i: (i, 0))
    full = lambda shape: pl.BlockSpec(shape, lambda i: tuple(0 for _ in shape))
    return pl.pallas_call(
        _epilogue_body,
        grid=grid,
        in_specs=[row_spec, row_spec,
                  row_spec, full((C, C)), full((C,)), full((C, C)), full((C,)),
                  full((C, C)), full((C,)), full((C, 1))],
        out_specs=row_spec,
        out_shape=jax.ShapeDtypeStruct((n, C), jnp.float32),
    )(h1, h2, h3, l1_W, l1_b, l2_W, l2_b, l3_W, l3_b, agg_W)


def kernel(x, edge_index, edge_index_2_hop, edge_index_knn,
           g1_W, g1_as, g1_ad, g1_b, g2_W, g2_as, g2_ad, g2_b,
           g3_W, g3_as, g3_ad, g3_b, g4_W, g4_as, g4_ad, g4_b,
           g5_W, g5_as, g5_ad, g5_b, g6_W, g6_as, g6_ad, g6_b,
           l1_W, l1_b, l2_W, l2_b, l3_W, l3_b, agg_W):
    h1 = _gat(x, edge_index, g1_W, g1_as, g1_ad, g1_b, HID)
    h1 = jax.nn.relu(h1)
    h1 = _gat(h1, edge_index, g2_W, g2_as, g2_ad, g2_b, C)
    h2 = _gat(x, edge_index_2_hop, g3_W, g3_as, g3_ad, g3_b, HID)
    h2 = jax.nn.relu(h2)
    h2 = _gat(h2, edge_index_2_hop, g4_W, g4_as, g4_ad, g4_b, C)
    h3 = _gat(x, edge_index_knn, g5_W, g5_as, g5_ad, g5_b, HID)
    h3 = jax.nn.relu(h3)
    h3 = _gat(h3, edge_index_knn, g6_W, g6_as, g6_ad, g6_b, C)
    return _epilogue(h1, h2, h3, l1_W, l1_b, l2_W, l2_b, l3_W, l3_b, agg_W)


# speedup vs baseline: 8.0374x; 7.8192x over previous
import functools

import jax
import jax.numpy as jnp
from jax import lax
from jax.experimental import pallas as pl
from jax.experimental.pallas import tpu as pltpu
from jax.experimental.pallas import tpu_sc as plsc

N = 10000
D = 256
HID = 128
C = 40
H = 8

NP = 10240
NCORE = 2
NSUB = 16
CH = 128
RPS = NP // NSUB
CP = 128
D8 = NP // 8
BLK = 256


def _prep_body(f, x_ref, W_ref, as_ref, ad_ref, hpad_ref, asrc_ref,
               adst_ref):
    x = x_ref[...]
    h = jnp.dot(x, W_ref[...], preferred_element_type=jnp.float32)
    b = x.shape[0]
    h3 = h.reshape(b, H, f)
    asrc_ref[...] = jnp.sum(h3 * as_ref[...][None], axis=-1)
    adst_ref[...] = jnp.sum(h3 * ad_ref[...][None], axis=-1)
    if f < 128:
        h3 = jnp.concatenate(
            [h3, jnp.zeros((b, H, 128 - f), jnp.float32)], axis=-1)
    hpad_ref[...] = h3.reshape(b * H, 128)


def _prep(xp, W, a_s, a_d, f):
    din = xp.shape[1]
    grid = (NP // BLK,)
    return pl.pallas_call(
        functools.partial(_prep_body, f),
        grid=grid,
        in_specs=[
            pl.BlockSpec((BLK, din), lambda i: (i, 0)),
            pl.BlockSpec((din, H * f), lambda i: (0, 0)),
            pl.BlockSpec((H, f), lambda i: (0, 0)),
            pl.BlockSpec((H, f), lambda i: (0, 0)),
        ],
        out_specs=[
            pl.BlockSpec((BLK * H, 128), lambda i: (i, 0)),
            pl.BlockSpec((BLK, H), lambda i: (i, 0)),
            pl.BlockSpec((BLK, H), lambda i: (i, 0)),
        ],
        out_shape=[
            jax.ShapeDtypeStruct((NP * H, 128), jnp.float32),
            jax.ShapeDtypeStruct((NP, H), jnp.float32),
            jax.ShapeDtypeStruct((NP, H), jnp.float32),
        ],
    )(xp, W, a_s, a_d)


def _sc_body_hid(ew, hpad, asrc_f, adst_f, srcp, dstp, num_out, den_out,
                 src_v, dst_v, gidx_v, aidx_v, didx_v, dr8_v, onew_v, oold_v,
                 asg_v, adg_v, ex_v, rows_v, dorow_v, acc, accd):
    cid = lax.axis_index("c")
    sid = lax.axis_index("s")
    row0 = sid * RPS
    drow0 = sid * (D8 // NSUB)
    G = CH // 16
    z16 = jnp.zeros((16,), jnp.float32)
    lane0 = lax.iota(jnp.int32, 16) == 0

    def zdrow(i, _):
        for k in range(128 // 16):
            dorow_v[i, pl.ds(k * 16, 16)] = z16
        return 0
    lax.fori_loop(0, CH, zdrow, 0)
    for j in range((CH + 16) // 16):
        oold_v[pl.ds(j * 16, 16)] = jnp.zeros((16,), jnp.int32)

    def head_pass(hh, _):
        hd = cid * (H // NCORE) + hh
        hoff = hd * NP

        def zrow(i, _):
            for k in range(128 // 16):
                rows_v[i, pl.ds(k * 16, 16)] = z16
            return 0
        lax.fori_loop(0, CH, zrow, 0)

        def init_chunk(cc, _):
            pltpu.sync_copy(rows_v, acc.at[pl.ds(row0 + cc * CP, CP)])
            return 0
        lax.fori_loop(0, RPS // CP, init_chunk, 0)
        pltpu.sync_copy(rows_v.at[pl.ds(0, D8 // NSUB)],
                        accd.at[pl.ds(drow0, D8 // NSUB)])
        plsc.subcore_barrier()

        base = sid * ew

        def edge_chunk(cc, _):
            e0 = base + cc * CH
            pltpu.sync_copy(srcp.at[pl.ds(e0, CH)], src_v)
            pltpu.sync_copy(dstp.at[pl.ds(e0, CH)], dst_v)

            def idx16(j, _):
                o = j * 16
                si = src_v[pl.ds(o, 16)]
                di = dst_v[pl.ds(o, 16)]
                aidx_v[pl.ds(o, 16)] = si + hoff
                didx_v[pl.ds(o, 16)] = di + hoff
                gidx_v[pl.ds(o, 16)] = si * H + hd
                dr8_v[pl.ds(o, 16)] = lax.shift_right_logical(di, 3)
                onew_v[pl.ds(o, 16)] = lax.bitwise_and(di, 7) * 16
                return 0
            lax.fori_loop(0, G, idx16, 0)

            pltpu.sync_copy(asrc_f.at[aidx_v], asg_v)
            pltpu.sync_copy(adst_f.at[didx_v], adg_v)
            pltpu.sync_copy(hpad.at[gidx_v], rows_v)

            def ex16(j, _):
                o = j * 16
                v = asg_v[pl.ds(o, 16)] + adg_v[pl.ds(o, 16)]
                v = jnp.where(v >= 0.0, v, 0.2 * v)
                ex_v[pl.ds(o, 16)] = jnp.exp(v)
                return 0
            lax.fori_loop(0, G, ex16, 0)

            def scale_one(i, _):
                exb = ex_v[pl.ds(i, 16)][0]
                for k in range(128 // 16):
                    rows_v[i, pl.ds(k * 16, 16)] = (
                        rows_v[i, pl.ds(k * 16, 16)] * exb)
                po = oold_v[pl.ds(i, 16)][0]
                o = onew_v[pl.ds(i, 16)][0]
                dorow_v[i, pl.ds(po, 16)] = z16
                dorow_v[i, pl.ds(o, 16)] = jnp.where(lane0, exb, 0.0)
                return 0
            lax.fori_loop(0, CH, scale_one, 0)

            for j in range(G):
                oold_v[pl.ds(j * 16, 16)] = onew_v[pl.ds(j * 16, 16)]

            pltpu.sync_copy(rows_v, acc.at[dst_v], add=True)
            pltpu.sync_copy(dorow_v, accd.at[dr8_v], add=True)
            return 0
        lax.fori_loop(0, ew // CH, edge_chunk, 0)
        plsc.subcore_barrier()

        pltpu.sync_copy(acc.at[pl.ds(row0, RPS)],
                        num_out.at[hd, pl.ds(row0, RPS)])
        pltpu.sync_copy(accd.at[pl.ds(drow0, D8 // NSUB)],
                        den_out.at[hd, pl.ds(drow0, D8 // NSUB)])
        return 0
    lax.fori_loop(0, H // NCORE, head_pass, 0)


def _sc_aggregate_hid(hpad, asrc_f, adst_f, srcp, dstp):
    ep = srcp.shape[0]
    ew = ep // NSUB
    mesh = plsc.VectorSubcoreMesh(core_axis_name="c", subcore_axis_name="s")
    kern = pl.kernel(
        functools.partial(_sc_body_hid, ew),
        out_type=[
            jax.ShapeDtypeStruct((H, NP, 128), jnp.float32),
            jax.ShapeDtypeStruct((H, D8, 128), jnp.float32),
        ],
        mesh=mesh,
        scratch_types=[
            pltpu.VMEM((CH,), jnp.int32),
            pltpu.VMEM((CH,), jnp.int32),
            pltpu.VMEM((CH,), jnp.int32),
            pltpu.VMEM((CH,), jnp.int32),
            pltpu.VMEM((CH,), jnp.int32),
            pltpu.VMEM((CH,), jnp.int32),
            pltpu.VMEM((CH + 16,), jnp.int32),
            pltpu.VMEM((CH + 16,), jnp.int32),
            pltpu.VMEM((CH,), jnp.float32),
            pltpu.VMEM((CH,), jnp.float32),
            pltpu.VMEM((CH + 16,), jnp.float32),
            pltpu.VMEM((CH, 128), jnp.float32),
            pltpu.VMEM((CH, 128), jnp.float32),
            pltpu.VMEM_SHARED((NP, 128), jnp.float32),
            pltpu.VMEM_SHARED((D8, 128), jnp.float32),
        ],
    )
    return kern(hpad, asrc_f, adst_f, srcp, dstp)


def _finish_hid_body(num_ref, den_ref, b_ref, out_ref):
    num = num_ref[...]
    den = den_ref[...] + 1e-16
    o = jnp.sum(num / den[:, :, None], axis=0) * (1.0 / H) + b_ref[...]
    out_ref[...] = jnp.maximum(o, 0.0)


def _finish_hid(num, den, bias):
    grid = (NP // BLK,)
    return pl.pallas_call(
        _finish_hid_body,
        grid=grid,
        in_specs=[
            pl.BlockSpec((H, BLK, 128), lambda i: (0, i, 0)),
            pl.BlockSpec((H, BLK), lambda i: (0, i)),
            pl.BlockSpec((HID,), lambda i: (0,)),
        ],
        out_specs=pl.BlockSpec((BLK, HID), lambda i: (i, 0)),
        out_shape=jax.ShapeDtypeStruct((NP, HID), jnp.float32),
    )(num, den, bias)


def _sc_body_c(ew, hpad, asrc_f, adst_f, srcp, dstp, out,
               src_v, dst_v, gidx_v, aidx_v, didx_v, asg_v, adg_v, ex_v,
               rows_v, acc):
    cid = lax.axis_index("c")
    sid = lax.axis_index("s")
    row0 = sid * RPS
    G = CH // 16
    lane0 = lax.iota(jnp.int32, 16) == 0

    def head_pass(hh, _):
        hd = cid * (H // NCORE) + hh
        hoff = hd * NP

        def zrow(i, _):
            for k in range(128 // 16):
                rows_v[i, pl.ds(k * 16, 16)] = jnp.zeros((16,), jnp.float32)
            return 0
        lax.fori_loop(0, CH, zrow, 0)

        def init_chunk(cc, _):
            pltpu.sync_copy(rows_v, acc.at[pl.ds(row0 + cc * CP, CP)])
            return 0
        lax.fori_loop(0, RPS // CP, init_chunk, 0)
        plsc.subcore_barrier()

        base = sid * ew

        def edge_chunk(cc, _):
            e0 = base + cc * CH
            pltpu.sync_copy(srcp.at[pl.ds(e0, CH)], src_v)
            pltpu.sync_copy(dstp.at[pl.ds(e0, CH)], dst_v)

            def idx16(j, _):
                o = j * 16
                si = src_v[pl.ds(o, 16)]
                di = dst_v[pl.ds(o, 16)]
                aidx_v[pl.ds(o, 16)] = si + hoff
                didx_v[pl.ds(o, 16)] = di + hoff
                gidx_v[pl.ds(o, 16)] = si * H + hd
                return 0
            lax.fori_loop(0, G, idx16, 0)

            pltpu.sync_copy(asrc_f.at[aidx_v], asg_v)
            pltpu.sync_copy(adst_f.at[didx_v], adg_v)
            pltpu.sync_copy(hpad.at[gidx_v], rows_v)

            def ex16(j, _):
                o = j * 16
                v = asg_v[pl.ds(o, 16)] + adg_v[pl.ds(o, 16)]
                v = jnp.where(v >= 0.0, v, 0.2 * v)
                ex_v[pl.ds(o, 16)] = jnp.exp(v)
                return 0
            lax.fori_loop(0, G, ex16, 0)

            def scale_one(i, _):
                exb = ex_v[pl.ds(i, 16)][0]
                for k in range(3):
                    rows_v[i, pl.ds(k * 16, 16)] = (
                        rows_v[i, pl.ds(k * 16, 16)] * exb)
                rows_v[i, pl.ds(48, 16)] = jnp.where(lane0, exb, 0.0)
                return 0
            lax.fori_loop(0, CH, scale_one, 0)

            pltpu.sync_copy(rows_v, acc.at[dst_v], add=True)
            return 0
        lax.fori_loop(0, ew // CH, edge_chunk, 0)
        plsc.subcore_barrier()

        pltpu.sync_copy(acc.at[pl.ds(row0, RPS)],
                        out.at[hd, pl.ds(row0, RPS)])
        return 0
    lax.fori_loop(0, H // NCORE, head_pass, 0)


def _sc_aggregate_c(hpad, asrc_f, adst_f, srcp, dstp):
    ep = srcp.shape[0]
    ew = ep // NSUB
    mesh = plsc.VectorSubcoreMesh(core_axis_name="c", subcore_axis_name="s")
    kern = pl.kernel(
        functools.partial(_sc_body_c, ew),
        out_type=jax.ShapeDtypeStruct((H, NP, 128), jnp.float32),
        mesh=mesh,
        scratch_types=[
            pltpu.VMEM((CH,), jnp.int32),
            pltpu.VMEM((CH,), jnp.int32),
            pltpu.VMEM((CH,), jnp.int32),
            pltpu.VMEM((CH,), jnp.int32),
            pltpu.VMEM((CH,), jnp.int32),
            pltpu.VMEM((CH,), jnp.float32),
            pltpu.VMEM((CH,), jnp.float32),
            pltpu.VMEM((CH + 16,), jnp.float32),
            pltpu.VMEM((CH, 128), jnp.float32),
            pltpu.VMEM_SHARED((NP, 128), jnp.float32),
        ],
    )
    return kern(hpad, asrc_f, adst_f, srcp, dstp)


def _finish_c_body(sc_ref, b_ref, out_ref):
    blk = sc_ref[...]
    num = blk[:, :, 0:C]
    den = blk[:, :, 48] + 1e-16
    o = jnp.sum(num / den[:, :, None], axis=0) * (1.0 / H) + b_ref[...]
    out_ref[...] = o


def _finish_c(sc_out, bias):
    grid = (NP // BLK,)
    return pl.pallas_call(
        _finish_c_body,
        grid=grid,
        in_specs=[
            pl.BlockSpec((H, BLK, 128), lambda i: (0, i, 0)),
            pl.BlockSpec((C,), lambda i: (0,)),
        ],
        out_specs=pl.BlockSpec((BLK, C), lambda i: (i, 0)),
        out_shape=jax.ShapeDtypeStruct((NP, C), jnp.float32),
    )(sc_out, bias)


def _gat_layer_hid(xp, ei_pack, W, a_s, a_d, b):
    srcp, dstp = ei_pack
    hpad, asrc, adst = _prep(xp, W, a_s, a_d, HID)
    asrc_f = asrc.T.reshape(H * NP)
    adst_f = adst.T.reshape(H * NP)
    num, den = _sc_aggregate_hid(hpad, asrc_f, adst_f, srcp, dstp)
    den = den[:, :, 0:128:16].reshape(H, NP)
    return _finish_hid(num, den, b)


def _gat_layer_c(xp, ei_pack, W, a_s, a_d, b):
    srcp, dstp = ei_pack
    hpad, asrc, adst = _prep(xp, W, a_s, a_d, C)
    asrc_f = asrc.T.reshape(H * NP)
    adst_f = adst.T.reshape(H * NP)
    sc_out = _sc_aggregate_c(hpad, asrc_f, adst_f, srcp, dstp)
    return _finish_c(sc_out, b)


def _pad_edges(ei):
    src = jnp.concatenate([ei[0], jnp.arange(N, dtype=ei.dtype)])
    dst = jnp.concatenate([ei[1], jnp.arange(N, dtype=ei.dtype)])
    e = src.shape[0]
    ep = ((e + 2047) // 2048) * 2048
    pad = ep - e
    src = jnp.concatenate([src, jnp.zeros((pad,), ei.dtype)])
    dst = jnp.concatenate([dst, jnp.full((pad,), N, ei.dtype)])
    return src, dst


def _epilogue_body(h1_ref, h2_ref, h3_ref, l1W_ref, l1b_ref, l2W_ref, l2b_ref,
                   l3W_ref, l3b_ref, aggW_ref, out_ref):
    h1 = h1_ref[...]
    h2 = h2_ref[...]
    h3 = h3_ref[...]
    aggW = aggW_ref[...]
    a1 = jnp.tanh(h1 @ l1W_ref[...] + l1b_ref[...]) @ aggW
    a2 = jnp.tanh(h2 @ l2W_ref[...] + l2b_ref[...]) @ aggW
    a3 = jnp.tanh(h3 @ l3W_ref[...] + l3b_ref[...]) @ aggW
    m = jnp.maximum(jnp.maximum(a1, a2), a3)
    e1 = jnp.exp(a1 - m)
    e2 = jnp.exp(a2 - m)
    e3 = jnp.exp(a3 - m)
    tot = e1 + e2 + e3
    h = (e1 * h1 + e2 * h2 + e3 * h3) / tot
    hm = jnp.max(h, axis=1, keepdims=True)
    lse = jnp.log(jnp.sum(jnp.exp(h - hm), axis=1, keepdims=True)) + hm
    out_ref[...] = h - lse


def _epilogue(h1, h2, h3, l1_W, l1_b, l2_W, l2_b, l3_W, l3_b, agg_W):
    n = h1.shape[0]
    blk = 1000
    grid = (n // blk,)
    row_spec = pl.BlockSpec((blk, C), lambda i: (i, 0))
    full = lambda shape: pl.BlockSpec(shape, lambda i: tuple(0 for _ in shape))
    return pl.pallas_call(
        _epilogue_body,
        grid=grid,
        in_specs=[row_spec, row_spec,
                  row_spec, full((C, C)), full((C,)), full((C, C)), full((C,)),
                  full((C, C)), full((C,)), full((C, 1))],
        out_specs=row_spec,
        out_shape=jax.ShapeDtypeStruct((n, C), jnp.float32),
    )(h1, h2, h3, l1_W, l1_b, l2_W, l2_b, l3_W, l3_b, agg_W)


def kernel(x, edge_index, edge_index_2_hop, edge_index_knn,
           g1_W, g1_as, g1_ad, g1_b, g2_W, g2_as, g2_ad, g2_b,
           g3_W, g3_as, g3_ad, g3_b, g4_W, g4_as, g4_ad, g4_b,
           g5_W, g5_as, g5_ad, g5_b, g6_W, g6_as, g6_ad, g6_b,
           l1_W, l1_b, l2_W, l2_b, l3_W, l3_b, agg_W):
    xp = jnp.pad(x, ((0, NP - N), (0, 0)))
    e1 = _pad_edges(edge_index)
    e2 = _pad_edges(edge_index_2_hop)
    ek = _pad_edges(edge_index_knn)

    h1 = _gat_layer_hid(xp, e1, g1_W, g1_as, g1_ad, g1_b)
    h1 = _gat_layer_c(h1, e1, g2_W, g2_as, g2_ad, g2_b)
    h2 = _gat_layer_hid(xp, e2, g3_W, g3_as, g3_ad, g3_b)
    h2 = _gat_layer_c(h2, e2, g4_W, g4_as, g4_ad, g4_b)
    h3 = _gat_layer_hid(xp, ek, g5_W, g5_as, g5_ad, g5_b)
    h3 = _gat_layer_c(h3, ek, g6_W, g6_as, g6_ad, g6_b)

    return _epilogue(h1[:N], h2[:N], h3[:N],
                     l1_W, l1_b, l2_W, l2_b, l3_W, l3_b, agg_W)


# speedup vs baseline: 10.4946x; 1.3057x over previous
import functools

import jax
import jax.numpy as jnp
from jax import lax
from jax.experimental import pallas as pl
from jax.experimental.pallas import tpu as pltpu
from jax.experimental.pallas import tpu_sc as plsc

N = 10000
D = 256
HID = 128
C = 40
H = 8

NP = 10240
NCORE = 2
NSUB = 16
CH = 128
RPS = NP // NSUB
CP = 128
D8 = NP // 8
BLK = 256


def _prep_body(f, x_ref, W_ref, as_ref, ad_ref, hpad_ref, asrc_ref,
               adst_ref):
    x = x_ref[...]
    h = jnp.dot(x, W_ref[...], preferred_element_type=jnp.float32)
    b = x.shape[0]
    h3 = h.reshape(b, H, f)
    asrc_ref[...] = jnp.sum(h3 * as_ref[...][None], axis=-1)
    adst_ref[...] = jnp.sum(h3 * ad_ref[...][None], axis=-1)
    if f < 128:
        h3 = jnp.concatenate(
            [h3, jnp.zeros((b, H, 128 - f), jnp.float32)], axis=-1)
    hpad_ref[...] = h3.reshape(b * H, 128)


def _prep(xp, W, a_s, a_d, f):
    din = xp.shape[1]
    grid = (NP // BLK,)
    return pl.pallas_call(
        functools.partial(_prep_body, f),
        grid=grid,
        in_specs=[
            pl.BlockSpec((BLK, din), lambda i: (i, 0)),
            pl.BlockSpec((din, H * f), lambda i: (0, 0)),
            pl.BlockSpec((H, f), lambda i: (0, 0)),
            pl.BlockSpec((H, f), lambda i: (0, 0)),
        ],
        out_specs=[
            pl.BlockSpec((BLK * H, 128), lambda i: (i, 0)),
            pl.BlockSpec((BLK, H), lambda i: (i, 0)),
            pl.BlockSpec((BLK, H), lambda i: (i, 0)),
        ],
        out_shape=[
            jax.ShapeDtypeStruct((NP * H, 128), jnp.float32),
            jax.ShapeDtypeStruct((NP, H), jnp.float32),
            jax.ShapeDtypeStruct((NP, H), jnp.float32),
        ],
    )(xp, W, a_s, a_d)


def _sc_body_hid(ew, hpad, asrc_f, adst_f, srcp, dstp, num_out, den_out,
                 src_v, dst_v, gidx_v, aidx_v, didx_v, dr8_v, onew_v, oold_v,
                 asg_v, adg_v, ex_v, rows_v, dorow_v, acc, accd, sem):
    cid = lax.axis_index("c")
    sid = lax.axis_index("s")
    row0 = sid * RPS
    drow0 = sid * (D8 // NSUB)
    G = CH // 16
    z16 = jnp.zeros((16,), jnp.float32)
    lane0 = lax.iota(jnp.int32, 16) == 0

    def zdrow(i, _):
        for k in range(128 // 16):
            dorow_v[i, pl.ds(k * 16, 16)] = z16
        return 0
    lax.fori_loop(0, CH, zdrow, 0)
    for j in range((CH + 16) // 16):
        oold_v[pl.ds(j * 16, 16)] = jnp.zeros((16,), jnp.int32)

    def head_pass(hh, _):
        hd = cid * (H // NCORE) + hh
        hoff = hd * NP

        def zrow(i, _):
            for k in range(128 // 16):
                rows_v[i, pl.ds(k * 16, 16)] = z16
            return 0
        lax.fori_loop(0, CH, zrow, 0)

        def init_chunk(cc, _):
            pltpu.sync_copy(rows_v, acc.at[pl.ds(row0 + cc * CP, CP)])
            return 0
        lax.fori_loop(0, RPS // CP, init_chunk, 0)
        pltpu.sync_copy(rows_v.at[pl.ds(0, D8 // NSUB)],
                        accd.at[pl.ds(drow0, D8 // NSUB)])
        plsc.subcore_barrier()

        base = sid * ew

        def edge_chunk(cc, _):
            e0 = base + cc * CH
            h1 = pltpu.async_copy(srcp.at[pl.ds(e0, CH)], src_v, sem)
            h2 = pltpu.async_copy(dstp.at[pl.ds(e0, CH)], dst_v, sem)
            h1.wait()
            h2.wait()

            def idx16(j, _):
                o = j * 16
                si = src_v[pl.ds(o, 16)]
                di = dst_v[pl.ds(o, 16)]
                aidx_v[pl.ds(o, 16)] = si + hoff
                didx_v[pl.ds(o, 16)] = di + hoff
                gidx_v[pl.ds(o, 16)] = si * H + hd
                dr8_v[pl.ds(o, 16)] = lax.shift_right_logical(di, 3)
                onew_v[pl.ds(o, 16)] = lax.bitwise_and(di, 7) * 16
                return 0
            lax.fori_loop(0, G, idx16, 0)

            g1 = pltpu.async_copy(asrc_f.at[aidx_v], asg_v, sem)
            g2 = pltpu.async_copy(adst_f.at[didx_v], adg_v, sem)
            g3 = pltpu.async_copy(hpad.at[gidx_v], rows_v, sem)
            g1.wait()
            g2.wait()
            g3.wait()

            def ex16(j, _):
                o = j * 16
                v = asg_v[pl.ds(o, 16)] + adg_v[pl.ds(o, 16)]
                v = jnp.where(v >= 0.0, v, 0.2 * v)
                ex_v[pl.ds(o, 16)] = jnp.exp(v)
                return 0
            lax.fori_loop(0, G, ex16, 0)

            def scale_one(i, _):
                exb = ex_v[pl.ds(i, 16)][0]
                for k in range(128 // 16):
                    rows_v[i, pl.ds(k * 16, 16)] = (
                        rows_v[i, pl.ds(k * 16, 16)] * exb)
                po = oold_v[pl.ds(i, 16)][0]
                o = onew_v[pl.ds(i, 16)][0]
                dorow_v[i, pl.ds(po, 16)] = z16
                dorow_v[i, pl.ds(o, 16)] = jnp.where(lane0, exb, 0.0)
                return 0
            lax.fori_loop(0, CH, scale_one, 0)

            for j in range(G):
                oold_v[pl.ds(j * 16, 16)] = onew_v[pl.ds(j * 16, 16)]

            s1 = pltpu.async_copy(rows_v, acc.at[dst_v], sem, add=True)
            s2 = pltpu.async_copy(dorow_v, accd.at[dr8_v], sem, add=True)
            s1.wait()
            s2.wait()
            return 0
        lax.fori_loop(0, ew // CH, edge_chunk, 0)
        plsc.subcore_barrier()

        pltpu.sync_copy(acc.at[pl.ds(row0, RPS)],
                        num_out.at[hd, pl.ds(row0, RPS)])
        pltpu.sync_copy(accd.at[pl.ds(drow0, D8 // NSUB)],
                        den_out.at[hd, pl.ds(drow0, D8 // NSUB)])
        return 0
    lax.fori_loop(0, H // NCORE, head_pass, 0)


def _sc_aggregate_hid(hpad, asrc_f, adst_f, srcp, dstp):
    ep = srcp.shape[0]
    ew = ep // NSUB
    mesh = plsc.VectorSubcoreMesh(core_axis_name="c", subcore_axis_name="s")
    kern = pl.kernel(
        functools.partial(_sc_body_hid, ew),
        out_type=[
            jax.ShapeDtypeStruct((H, NP, 128), jnp.float32),
            jax.ShapeDtypeStruct((H, D8, 128), jnp.float32),
        ],
        mesh=mesh,
        scratch_types=[
            pltpu.VMEM((CH,), jnp.int32),
            pltpu.VMEM((CH,), jnp.int32),
            pltpu.VMEM((CH,), jnp.int32),
            pltpu.VMEM((CH,), jnp.int32),
            pltpu.VMEM((CH,), jnp.int32),
            pltpu.VMEM((CH,), jnp.int32),
            pltpu.VMEM((CH + 16,), jnp.int32),
            pltpu.VMEM((CH + 16,), jnp.int32),
            pltpu.VMEM((CH,), jnp.float32),
            pltpu.VMEM((CH,), jnp.float32),
            pltpu.VMEM((CH + 16,), jnp.float32),
            pltpu.VMEM((CH, 128), jnp.float32),
            pltpu.VMEM((CH, 128), jnp.float32),
            pltpu.VMEM_SHARED((NP, 128), jnp.float32),
            pltpu.VMEM_SHARED((D8, 128), jnp.float32),
            pltpu.SemaphoreType.DMA,
        ],
    )
    return kern(hpad, asrc_f, adst_f, srcp, dstp)


def _finish_hid_body(num_ref, den_ref, b_ref, out_ref):
    num = num_ref[...]
    den = den_ref[...] + 1e-16
    o = jnp.sum(num / den[:, :, None], axis=0) * (1.0 / H) + b_ref[...]
    out_ref[...] = jnp.maximum(o, 0.0)


def _finish_hid(num, den, bias):
    grid = (NP // BLK,)
    return pl.pallas_call(
        _finish_hid_body,
        grid=grid,
        in_specs=[
            pl.BlockSpec((H, BLK, 128), lambda i: (0, i, 0)),
            pl.BlockSpec((H, BLK), lambda i: (0, i)),
            pl.BlockSpec((HID,), lambda i: (0,)),
        ],
        out_specs=pl.BlockSpec((BLK, HID), lambda i: (i, 0)),
        out_shape=jax.ShapeDtypeStruct((NP, HID), jnp.float32),
    )(num, den, bias)


def _sc_body_c(ew, hpad, asrc_f, adst_f, srcp, dstp, out,
               src_v, dst_v, gidx_v, aidx_v, didx_v, asg_v, adg_v, ex_v,
               rows_v, acc, sem):
    cid = lax.axis_index("c")
    sid = lax.axis_index("s")
    row0 = sid * RPS
    G = CH // 16
    lane0 = lax.iota(jnp.int32, 16) == 0

    def head_pass(hh, _):
        hd = cid * (H // NCORE) + hh
        hoff = hd * NP

        def zrow(i, _):
            for k in range(128 // 16):
                rows_v[i, pl.ds(k * 16, 16)] = jnp.zeros((16,), jnp.float32)
            return 0
        lax.fori_loop(0, CH, zrow, 0)

        def init_chunk(cc, _):
            pltpu.sync_copy(rows_v, acc.at[pl.ds(row0 + cc * CP, CP)])
            return 0
        lax.fori_loop(0, RPS // CP, init_chunk, 0)
        plsc.subcore_barrier()

        base = sid * ew

        def edge_chunk(cc, _):
            e0 = base + cc * CH
            h1 = pltpu.async_copy(srcp.at[pl.ds(e0, CH)], src_v, sem)
            h2 = pltpu.async_copy(dstp.at[pl.ds(e0, CH)], dst_v, sem)
            h1.wait()
            h2.wait()

            def idx16(j, _):
                o = j * 16
                si = src_v[pl.ds(o, 16)]
                di = dst_v[pl.ds(o, 16)]
                aidx_v[pl.ds(o, 16)] = si + hoff
                didx_v[pl.ds(o, 16)] = di + hoff
                gidx_v[pl.ds(o, 16)] = si * H + hd
                return 0
            lax.fori_loop(0, G, idx16, 0)

            g1 = pltpu.async_copy(asrc_f.at[aidx_v], asg_v, sem)
            g2 = pltpu.async_copy(adst_f.at[didx_v], adg_v, sem)
            g3 = pltpu.async_copy(hpad.at[gidx_v], rows_v, sem)
            g1.wait()
            g2.wait()
            g3.wait()

            def ex16(j, _):
                o = j * 16
                v = asg_v[pl.ds(o, 16)] + adg_v[pl.ds(o, 16)]
                v = jnp.where(v >= 0.0, v, 0.2 * v)
                ex_v[pl.ds(o, 16)] = jnp.exp(v)
                return 0
            lax.fori_loop(0, G, ex16, 0)

            def scale_one(i, _):
                exb = ex_v[pl.ds(i, 16)][0]
                for k in range(3):
                    rows_v[i, pl.ds(k * 16, 16)] = (
                        rows_v[i, pl.ds(k * 16, 16)] * exb)
                rows_v[i, pl.ds(48, 16)] = jnp.where(lane0, exb, 0.0)
                return 0
            lax.fori_loop(0, CH, scale_one, 0)

            pltpu.sync_copy(rows_v, acc.at[dst_v], add=True)
            return 0
        lax.fori_loop(0, ew // CH, edge_chunk, 0)
        plsc.subcore_barrier()

        pltpu.sync_copy(acc.at[pl.ds(row0, RPS)],
                        out.at[hd, pl.ds(row0, RPS)])
        return 0
    lax.fori_loop(0, H // NCORE, head_pass, 0)


def _sc_aggregate_c(hpad, asrc_f, adst_f, srcp, dstp):
    ep = srcp.shape[0]
    ew = ep // NSUB
    mesh = plsc.VectorSubcoreMesh(core_axis_name="c", subcore_axis_name="s")
    kern = pl.kernel(
        functools.partial(_sc_body_c, ew),
        out_type=jax.ShapeDtypeStruct((H, NP, 128), jnp.float32),
        mesh=mesh,
        scratch_types=[
            pltpu.VMEM((CH,), jnp.int32),
            pltpu.VMEM((CH,), jnp.int32),
            pltpu.VMEM((CH,), jnp.int32),
            pltpu.VMEM((CH,), jnp.int32),
            pltpu.VMEM((CH,), jnp.int32),
            pltpu.VMEM((CH,), jnp.float32),
            pltpu.VMEM((CH,), jnp.float32),
            pltpu.VMEM((CH + 16,), jnp.float32),
            pltpu.VMEM((CH, 128), jnp.float32),
            pltpu.VMEM_SHARED((NP, 128), jnp.float32),
            pltpu.SemaphoreType.DMA,
        ],
    )
    return kern(hpad, asrc_f, adst_f, srcp, dstp)


def _finish_c_body(sc_ref, b_ref, out_ref):
    blk = sc_ref[...]
    num = blk[:, :, 0:C]
    den = blk[:, :, 48] + 1e-16
    o = jnp.sum(num / den[:, :, None], axis=0) * (1.0 / H) + b_ref[...]
    out_ref[...] = o


def _finish_c(sc_out, bias):
    grid = (NP // BLK,)
    return pl.pallas_call(
        _finish_c_body,
        grid=grid,
        in_specs=[
            pl.BlockSpec((H, BLK, 128), lambda i: (0, i, 0)),
            pl.BlockSpec((C,), lambda i: (0,)),
        ],
        out_specs=pl.BlockSpec((BLK, C), lambda i: (i, 0)),
        out_shape=jax.ShapeDtypeStruct((NP, C), jnp.float32),
    )(sc_out, bias)


def _gat_layer_hid(xp, ei_pack, W, a_s, a_d, b):
    srcp, dstp = ei_pack
    hpad, asrc, adst = _prep(xp, W, a_s, a_d, HID)
    asrc_f = asrc.T.reshape(H * NP)
    adst_f = adst.T.reshape(H * NP)
    num, den = _sc_aggregate_hid(hpad, asrc_f, adst_f, srcp, dstp)
    den = den[:, :, 0:128:16].reshape(H, NP)
    return _finish_hid(num, den, b)


def _gat_layer_c(xp, ei_pack, W, a_s, a_d, b):
    srcp, dstp = ei_pack
    hpad, asrc, adst = _prep(xp, W, a_s, a_d, C)
    asrc_f = asrc.T.reshape(H * NP)
    adst_f = adst.T.reshape(H * NP)
    sc_out = _sc_aggregate_c(hpad, asrc_f, adst_f, srcp, dstp)
    return _finish_c(sc_out, b)


def _pad_edges(ei):
    src = jnp.concatenate([ei[0], jnp.arange(N, dtype=ei.dtype)])
    dst = jnp.concatenate([ei[1], jnp.arange(N, dtype=ei.dtype)])
    e = src.shape[0]
    ep = ((e + 2047) // 2048) * 2048
    pad = ep - e
    src = jnp.concatenate([src, jnp.zeros((pad,), ei.dtype)])
    dst = jnp.concatenate([dst, jnp.full((pad,), N, ei.dtype)])
    return src, dst


def _epilogue_body(h1_ref, h2_ref, h3_ref, l1W_ref, l1b_ref, l2W_ref, l2b_ref,
                   l3W_ref, l3b_ref, aggW_ref, out_ref):
    h1 = h1_ref[...]
    h2 = h2_ref[...]
    h3 = h3_ref[...]
    aggW = aggW_ref[...]
    a1 = jnp.tanh(h1 @ l1W_ref[...] + l1b_ref[...]) @ aggW
    a2 = jnp.tanh(h2 @ l2W_ref[...] + l2b_ref[...]) @ aggW
    a3 = jnp.tanh(h3 @ l3W_ref[...] + l3b_ref[...]) @ aggW
    m = jnp.maximum(jnp.maximum(a1, a2), a3)
    e1 = jnp.exp(a1 - m)
    e2 = jnp.exp(a2 - m)
    e3 = jnp.exp(a3 - m)
    tot = e1 + e2 + e3
    h = (e1 * h1 + e2 * h2 + e3 * h3) / tot
    hm = jnp.max(h, axis=1, keepdims=True)
    lse = jnp.log(jnp.sum(jnp.exp(h - hm), axis=1, keepdims=True)) + hm
    out_ref[...] = h - lse


def _epilogue(h1, h2, h3, l1_W, l1_b, l2_W, l2_b, l3_W, l3_b, agg_W):
    n = h1.shape[0]
    blk = 1000
    grid = (n // blk,)
    row_spec = pl.BlockSpec((blk, C), lambda i: (i, 0))
    full = lambda shape: pl.BlockSpec(shape, lambda i: tuple(0 for _ in shape))
    return pl.pallas_call(
        _epilogue_body,
        grid=grid,
        in_specs=[row_spec, row_spec,
                  row_spec, full((C, C)), full((C,)), full((C, C)), full((C,)),
                  full((C, C)), full((C,)), full((C, 1))],
        out_specs=row_spec,
        out_shape=jax.ShapeDtypeStruct((n, C), jnp.float32),
    )(h1, h2, h3, l1_W, l1_b, l2_W, l2_b, l3_W, l3_b, agg_W)


def kernel(x, edge_index, edge_index_2_hop, edge_index_knn,
           g1_W, g1_as, g1_ad, g1_b, g2_W, g2_as, g2_ad, g2_b,
           g3_W, g3_as, g3_ad, g3_b, g4_W, g4_as, g4_ad, g4_b,
           g5_W, g5_as, g5_ad, g5_b, g6_W, g6_as, g6_ad, g6_b,
           l1_W, l1_b, l2_W, l2_b, l3_W, l3_b, agg_W):
    xp = jnp.pad(x, ((0, NP - N), (0, 0)))
    e1 = _pad_edges(edge_index)
    e2 = _pad_edges(edge_index_2_hop)
    ek = _pad_edges(edge_index_knn)

    h1 = _gat_layer_hid(xp, e1, g1_W, g1_as, g1_ad, g1_b)
    h1 = _gat_layer_c(h1, e1, g2_W, g2_as, g2_ad, g2_b)
    h2 = _gat_layer_hid(xp, e2, g3_W, g3_as, g3_ad, g3_b)
    h2 = _gat_layer_c(h2, e2, g4_W, g4_as, g4_ad, g4_b)
    h3 = _gat_layer_hid(xp, ek, g5_W, g5_as, g5_ad, g5_b)
    h3 = _gat_layer_c(h3, ek, g6_W, g6_as, g6_ad, g6_b)

    return _epilogue(h1[:N], h2[:N], h3[:N],
                     l1_W, l1_b, l2_W, l2_b, l3_W, l3_b, agg_W)


# speedup vs baseline: 14.6339x; 1.3944x over previous
import functools

import jax
import jax.numpy as jnp
from jax import lax
from jax.experimental import pallas as pl
from jax.experimental.pallas import tpu as pltpu
from jax.experimental.pallas import tpu_sc as plsc

N = 10000
D = 256
HID = 128
C = 40
H = 8

NP = 10240
NCORE = 2
NSUB = 16
CHH = 64
RPS = NP // NSUB
D8 = NP // 8
BLK = 256


def _prep_body(f, x_ref, W_ref, as_ref, ad_ref, hpad_ref, asrc_ref,
               adst_ref):
    x = x_ref[...]
    h = jnp.dot(x, W_ref[...], preferred_element_type=jnp.float32)
    b = x.shape[0]
    h3 = h.reshape(b, H, f)
    asrc_ref[...] = jnp.sum(h3 * as_ref[...][None], axis=-1)
    adst_ref[...] = jnp.sum(h3 * ad_ref[...][None], axis=-1)
    if f < 128:
        h3 = jnp.concatenate(
            [h3, jnp.zeros((b, H, 128 - f), jnp.float32)], axis=-1)
    hpad_ref[...] = h3.reshape(b * H, 128)


def _prep(xp, W, a_s, a_d, f):
    din = xp.shape[1]
    grid = (NP // BLK,)
    return pl.pallas_call(
        functools.partial(_prep_body, f),
        grid=grid,
        in_specs=[
            pl.BlockSpec((BLK, din), lambda i: (i, 0)),
            pl.BlockSpec((din, H * f), lambda i: (0, 0)),
            pl.BlockSpec((H, f), lambda i: (0, 0)),
            pl.BlockSpec((H, f), lambda i: (0, 0)),
        ],
        out_specs=[
            pl.BlockSpec((BLK * H, 128), lambda i: (i, 0)),
            pl.BlockSpec((BLK, H), lambda i: (i, 0)),
            pl.BlockSpec((BLK, H), lambda i: (i, 0)),
        ],
        out_shape=[
            jax.ShapeDtypeStruct((NP * H, 128), jnp.float32),
            jax.ShapeDtypeStruct((NP, H), jnp.float32),
            jax.ShapeDtypeStruct((NP, H), jnp.float32),
        ],
    )(xp, W, a_s, a_d)


def _sc_edge_body(hid, ew, nc, hpad, asrc_f, adst_f, srcp, dstp, *rest):
    if hid:
        num_out, den_out = rest[0], rest[1]
        sc = rest[2:]
    else:
        num_out = rest[0]
        sc = rest[1:]
    (src2, dst2, gidx2, aidx2, didx2, asg2, adg2, ex2, rows2,
     sdst2) = [sc[2 * i:2 * i + 2] for i in range(10)]
    sc = sc[20:]
    if hid:
        dr82, onew2, oold2, dorow2, sdr82 = [
            sc[2 * i:2 * i + 2] for i in range(5)]
        acc, accd, esem, gsem, ssem = sc[10:]
    else:
        acc, esem, gsem, ssem = sc
        accd = None

    cid = lax.axis_index("c")
    sid = lax.axis_index("s")
    row0 = sid * RPS
    drow0 = sid * (D8 // NSUB)
    G = CHH // 16
    NV = 8 if hid else 3
    z16 = jnp.zeros((16,), jnp.float32)
    zi16 = jnp.zeros((16,), jnp.int32)
    lane0 = lax.iota(jnp.int32, 16) == 0

    if hid:
        for b in range(2):
            def zdrow(i, _, b=b):
                for k in range(8):
                    dorow2[b][i, pl.ds(k * 16, 16)] = z16
                return 0
            lax.fori_loop(0, CHH, zdrow, 0)
            for j in range((CHH + 16) // 16):
                oold2[b][pl.ds(j * 16, 16)] = zi16

    def head_pass(hh, _):
        hd = cid * (H // NCORE) + hh
        hoff = hd * NP

        def zrow(i, _):
            for k in range(8):
                rows2[0][i, pl.ds(k * 16, 16)] = z16
            return 0
        lax.fori_loop(0, CHH, zrow, 0)

        def init_chunk(cc, _):
            pltpu.sync_copy(rows2[0], acc.at[pl.ds(row0 + cc * CHH, CHH)])
            return 0
        lax.fori_loop(0, RPS // CHH, init_chunk, 0)
        if hid:
            pltpu.sync_copy(rows2[0].at[pl.ds(0, 64)],
                            accd.at[pl.ds(drow0, 64)])
            pltpu.sync_copy(rows2[0].at[pl.ds(0, 16)],
                            accd.at[pl.ds(drow0 + 64, 16)])
        plsc.subcore_barrier()

        base = sid * ew

        def comp_idx(b):
            def f(j, _):
                o = j * 16
                si = src2[b][pl.ds(o, 16)]
                di = dst2[b][pl.ds(o, 16)]
                aidx2[b][pl.ds(o, 16)] = si + hoff
                didx2[b][pl.ds(o, 16)] = di + hoff
                gidx2[b][pl.ds(o, 16)] = si * H + hd
                if hid:
                    dr82[b][pl.ds(o, 16)] = lax.shift_right_logical(di, 3)
                    onew2[b][pl.ds(o, 16)] = lax.bitwise_and(di, 7) * 16
                return 0
            lax.fori_loop(0, G, f, 0)

        def issue_gathers(b):
            pltpu.async_copy(asrc_f.at[aidx2[b]], asg2[b], gsem)
            pltpu.async_copy(adst_f.at[didx2[b]], adg2[b], gsem)
            pltpu.async_copy(hpad.at[gidx2[b]], rows2[b], gsem)

        def drain_gathers(b):
            pltpu.make_async_copy(asrc_f.at[aidx2[b]], asg2[b], gsem).wait()
            pltpu.make_async_copy(adst_f.at[didx2[b]], adg2[b], gsem).wait()
            pltpu.make_async_copy(hpad.at[gidx2[b]], rows2[b], gsem).wait()

        def issue_scatter(b):
            pltpu.async_copy(rows2[b], acc.at[sdst2[b]], ssem, add=True)
            if hid:
                pltpu.async_copy(dorow2[b], accd.at[sdr82[b]], ssem, add=True)

        def drain_scatter(b):
            pltpu.make_async_copy(rows2[b], acc.at[sdst2[b]], ssem).wait()
            if hid:
                pltpu.make_async_copy(
                    dorow2[b], accd.at[sdr82[b]], ssem).wait()

        def load_srcdst(b, e0):
            h1 = pltpu.async_copy(srcp.at[pl.ds(e0, CHH)], src2[b], esem)
            h2 = pltpu.async_copy(dstp.at[pl.ds(e0, CHH)], dst2[b], esem)
            return h1, h2

        def drain_srcdst(b, e0):
            pltpu.make_async_copy(
                srcp.at[pl.ds(e0, CHH)], src2[b], esem).wait()
            pltpu.make_async_copy(
                dstp.at[pl.ds(e0, CHH)], dst2[b], esem).wait()

        def compute(b):
            def ex16(j, _):
                o = j * 16
                v = asg2[b][pl.ds(o, 16)] + adg2[b][pl.ds(o, 16)]
                v = jnp.where(v >= 0.0, v, 0.2 * v)
                ex2[b][pl.ds(o, 16)] = jnp.exp(v)
                return 0
            lax.fori_loop(0, G, ex16, 0)

            def scale_one(i, _):
                exb = ex2[b][pl.ds(i, 16)][0]
                for k in range(NV):
                    rows2[b][i, pl.ds(k * 16, 16)] = (
                        rows2[b][i, pl.ds(k * 16, 16)] * exb)
                if hid:
                    po = oold2[b][pl.ds(i, 16)][0]
                    no = onew2[b][pl.ds(i, 16)][0]
                    dorow2[b][i, pl.ds(po, 16)] = z16
                    dorow2[b][i, pl.ds(no, 16)] = jnp.where(lane0, exb, 0.0)
                else:
                    rows2[b][i, pl.ds(48, 16)] = jnp.where(lane0, exb, 0.0)
                return 0
            lax.fori_loop(0, CHH, scale_one, 0)

            for j in range(G):
                o = j * 16
                sdst2[b][pl.ds(o, 16)] = dst2[b][pl.ds(o, 16)]
                if hid:
                    sdr82[b][pl.ds(o, 16)] = dr82[b][pl.ds(o, 16)]
                    oold2[b][pl.ds(o, 16)] = onew2[b][pl.ds(o, 16)]

        h1, h2 = load_srcdst(0, base)
        h1.wait()
        h2.wait()
        comp_idx(0)
        issue_gathers(0)
        load_srcdst(1, base + CHH)

        def pair_body(c2, _):
            for b in range(2):
                cc2 = c2 * 2 + b
                e0 = base + cc2 * CHH
                if b == 0:
                    @pl.when(c2 > 0)
                    def _():
                        drain_scatter(1)
                else:
                    drain_scatter(0)
                if b == 0:
                    drain_srcdst(1, e0 + CHH)
                    comp_idx(1)
                    issue_gathers(1)
                else:
                    @pl.when(c2 < nc // 2 - 1)
                    def _():
                        drain_srcdst(0, e0 + CHH)
                        comp_idx(0)
                        issue_gathers(0)
                drain_gathers(b)
                compute(b)
                issue_scatter(b)
                @pl.when(cc2 + 2 < nc)
                def _():
                    load_srcdst(b, e0 + 2 * CHH)
            return 0
        lax.fori_loop(0, nc // 2, pair_body, 0)
        drain_scatter(1)
        plsc.subcore_barrier()

        pltpu.sync_copy(acc.at[pl.ds(row0, RPS)],
                        num_out.at[hd, pl.ds(row0, RPS)])
        if hid:
            pltpu.sync_copy(accd.at[pl.ds(drow0, D8 // NSUB)],
                            den_out.at[hd, pl.ds(drow0, D8 // NSUB)])
        return 0
    lax.fori_loop(0, H // NCORE, head_pass, 0)


def _sc_aggregate(hpad, asrc_f, adst_f, srcp, dstp, hid):
    ep = srcp.shape[0]
    ew = ep // NSUB
    nc = ew // CHH
    mesh = plsc.VectorSubcoreMesh(core_axis_name="c", subcore_axis_name="s")
    if hid:
        out_type = [
            jax.ShapeDtypeStruct((H, NP, 128), jnp.float32),
            jax.ShapeDtypeStruct((H, D8, 128), jnp.float32),
        ]
    else:
        out_type = jax.ShapeDtypeStruct((H, NP, 128), jnp.float32)
    scratch = (
        [pltpu.VMEM((CHH,), jnp.int32)] * 2
        + [pltpu.VMEM((CHH,), jnp.int32)] * 2
        + [pltpu.VMEM((CHH,), jnp.int32)] * 2
        + [pltpu.VMEM((CHH,), jnp.int32)] * 2
        + [pltpu.VMEM((CHH,), jnp.int32)] * 2
        + [pltpu.VMEM((CHH,), jnp.float32)] * 2
        + [pltpu.VMEM((CHH,), jnp.float32)] * 2
        + [pltpu.VMEM((CHH + 16,), jnp.float32)] * 2
        + [pltpu.VMEM((CHH, 128), jnp.float32)] * 2
        + [pltpu.VMEM((CHH,), jnp.int32)] * 2
    )
    if hid:
        scratch += (
            [pltpu.VMEM((CHH,), jnp.int32)] * 2
            + [pltpu.VMEM((CHH + 16,), jnp.int32)] * 2
            + [pltpu.VMEM((CHH + 16,), jnp.int32)] * 2
            + [pltpu.VMEM((CHH, 128), jnp.float32)] * 2
            + [pltpu.VMEM((CHH,), jnp.int32)] * 2
            + [pltpu.VMEM_SHARED((NP, 128), jnp.float32),
               pltpu.VMEM_SHARED((D8, 128), jnp.float32)])
    else:
        scratch += [pltpu.VMEM_SHARED((NP, 128), jnp.float32)]
    scratch += [pltpu.SemaphoreType.DMA] * 3
    kern = pl.kernel(
        functools.partial(_sc_edge_body, hid, ew, nc),
        out_type=out_type,
        mesh=mesh,
        scratch_types=scratch,
    )
    return kern(hpad, asrc_f, adst_f, srcp, dstp)


def _finish_hid_body(num_ref, den_ref, b_ref, out_ref):
    num = num_ref[...]
    den = den_ref[...] + 1e-16
    o = jnp.sum(num / den[:, :, None], axis=0) * (1.0 / H) + b_ref[...]
    out_ref[...] = jnp.maximum(o, 0.0)


def _finish_hid(num, den, bias):
    grid = (NP // BLK,)
    return pl.pallas_call(
        _finish_hid_body,
        grid=grid,
        in_specs=[
            pl.BlockSpec((H, BLK, 128), lambda i: (0, i, 0)),
            pl.BlockSpec((H, BLK), lambda i: (0, i)),
            pl.BlockSpec((HID,), lambda i: (0,)),
        ],
        out_specs=pl.BlockSpec((BLK, HID), lambda i: (i, 0)),
        out_shape=jax.ShapeDtypeStruct((NP, HID), jnp.float32),
    )(num, den, bias)


def _finish_c_body(sc_ref, b_ref, out_ref):
    blk = sc_ref[...]
    num = blk[:, :, 0:C]
    den = blk[:, :, 48] + 1e-16
    o = jnp.sum(num / den[:, :, None], axis=0) * (1.0 / H) + b_ref[...]
    out_ref[...] = o


def _finish_c(sc_out, bias):
    grid = (NP // BLK,)
    return pl.pallas_call(
        _finish_c_body,
        grid=grid,
        in_specs=[
            pl.BlockSpec((H, BLK, 128), lambda i: (0, i, 0)),
            pl.BlockSpec((C,), lambda i: (0,)),
        ],
        out_specs=pl.BlockSpec((BLK, C), lambda i: (i, 0)),
        out_shape=jax.ShapeDtypeStruct((NP, C), jnp.float32),
    )(sc_out, bias)


def _gat_layer_hid(xp, ei_pack, W, a_s, a_d, b):
    srcp, dstp = ei_pack
    hpad, asrc, adst = _prep(xp, W, a_s, a_d, HID)
    asrc_f = asrc.T.reshape(H * NP)
    adst_f = adst.T.reshape(H * NP)
    num, den = _sc_aggregate(hpad, asrc_f, adst_f, srcp, dstp, True)
    den = den[:, :, 0:128:16].reshape(H, NP)
    return _finish_hid(num, den, b)


def _gat_layer_c(xp, ei_pack, W, a_s, a_d, b):
    srcp, dstp = ei_pack
    hpad, asrc, adst = _prep(xp, W, a_s, a_d, C)
    asrc_f = asrc.T.reshape(H * NP)
    adst_f = adst.T.reshape(H * NP)
    sc_out = _sc_aggregate(hpad, asrc_f, adst_f, srcp, dstp, False)
    return _finish_c(sc_out, b)


def _pad_edges(ei):
    src = jnp.concatenate([ei[0], jnp.arange(N, dtype=ei.dtype)])
    dst = jnp.concatenate([ei[1], jnp.arange(N, dtype=ei.dtype)])
    e = src.shape[0]
    ep = ((e + 2047) // 2048) * 2048
    pad = ep - e
    src = jnp.concatenate([src, jnp.zeros((pad,), ei.dtype)])
    dst = jnp.concatenate([dst, jnp.full((pad,), N, ei.dtype)])
    return src, dst


def _epilogue_body(h1_ref, h2_ref, h3_ref, l1W_ref, l1b_ref, l2W_ref, l2b_ref,
                   l3W_ref, l3b_ref, aggW_ref, out_ref):
    h1 = h1_ref[...]
    h2 = h2_ref[...]
    h3 = h3_ref[...]
    aggW = aggW_ref[...]
    a1 = jnp.tanh(h1 @ l1W_ref[...] + l1b_ref[...]) @ aggW
    a2 = jnp.tanh(h2 @ l2W_ref[...] + l2b_ref[...]) @ aggW
    a3 = jnp.tanh(h3 @ l3W_ref[...] + l3b_ref[...]) @ aggW
    m = jnp.maximum(jnp.maximum(a1, a2), a3)
    e1 = jnp.exp(a1 - m)
    e2 = jnp.exp(a2 - m)
    e3 = jnp.exp(a3 - m)
    tot = e1 + e2 + e3
    h = (e1 * h1 + e2 * h2 + e3 * h3) / tot
    hm = jnp.max(h, axis=1, keepdims=True)
    lse = jnp.log(jnp.sum(jnp.exp(h - hm), axis=1, keepdims=True)) + hm
    out_ref[...] = h - lse


def _epilogue(h1, h2, h3, l1_W, l1_b, l2_W, l2_b, l3_W, l3_b, agg_W):
    n = h1.shape[0]
    blk = 1000
    grid = (n // blk,)
    row_spec = pl.BlockSpec((blk, C), lambda i: (i, 0))
    full = lambda shape: pl.BlockSpec(shape, lambda i: tuple(0 for _ in shape))
    return pl.pallas_call(
        _epilogue_body,
        grid=grid,
        in_specs=[row_spec, row_spec,
                  row_spec, full((C, C)), full((C,)), full((C, C)), full((C,)),
                  full((C, C)), full((C,)), full((C, 1))],
        out_specs=row_spec,
        out_shape=jax.ShapeDtypeStruct((n, C), jnp.float32),
    )(h1, h2, h3, l1_W, l1_b, l2_W, l2_b, l3_W, l3_b, agg_W)


def kernel(x, edge_index, edge_index_2_hop, edge_index_knn,
           g1_W, g1_as, g1_ad, g1_b, g2_W, g2_as, g2_ad, g2_b,
           g3_W, g3_as, g3_ad, g3_b, g4_W, g4_as, g4_ad, g4_b,
           g5_W, g5_as, g5_ad, g5_b, g6_W, g6_as, g6_ad, g6_b,
           l1_W, l1_b, l2_W, l2_b, l3_W, l3_b, agg_W):
    xp = jnp.pad(x, ((0, NP - N), (0, 0)))
    e1 = _pad_edges(edge_index)
    e2 = _pad_edges(edge_index_2_hop)
    ek = _pad_edges(edge_index_knn)

    h1 = _gat_layer_hid(xp, e1, g1_W, g1_as, g1_ad, g1_b)
    h1 = _gat_layer_c(h1, e1, g2_W, g2_as, g2_ad, g2_b)
    h2 = _gat_layer_hid(xp, e2, g3_W, g3_as, g3_ad, g3_b)
    h2 = _gat_layer_c(h2, e2, g4_W, g4_as, g4_ad, g4_b)
    h3 = _gat_layer_hid(xp, ek, g5_W, g5_as, g5_ad, g5_b)
    h3 = _gat_layer_c(h3, ek, g6_W, g6_as, g6_ad, g6_b)

    return _epilogue(h1[:N], h2[:N], h3[:N],
                     l1_W, l1_b, l2_W, l2_b, l3_W, l3_b, agg_W)
